# baseline (speedup 1.0000x reference)
import functools

import jax
import jax.numpy as jnp
from jax import lax
from jax.experimental import pallas as pl
from jax.experimental.pallas import tpu as pltpu
from jax.experimental.pallas import tpu_sc as plsc

N = 10000
E = 320000
H = 128
NQ = 7
QL = 2
DIMQ = 2 ** NQ
NF = 100.0
NLAYERS = 2

BE = 2000
BN = 2000

NW = 32
EPW = E // NW
CH = 80
NCH = EPW // CH
NP = 10240
NPS = NP // 16
ZR = 80

NSUB = 16
EPS = E // NSUB
CH2 = 125
NCH2 = EPS // CH2
NLD = N // NSUB

_SC_MESH = plsc.VectorSubcoreMesh(core_axis_name="c", subcore_axis_name="s")


def _silu(v):
    return v * jax.nn.sigmoid(v)


def _apply_1q(st, gate, wire):
    B = st.shape[0]
    s = st.reshape(B, 2 ** wire, 2, -1)
    s = jnp.einsum('ab,slbr->slar', gate, s)
    return s.reshape(B, -1)


def _apply_cnot(st, c, t):
    B = st.shape[0]
    s = st.reshape((B,) + (2,) * NQ)
    s0 = jnp.take(s, jnp.array([0]), axis=c + 1)
    s1 = jnp.take(s, jnp.array([1]), axis=c + 1)
    s1 = jnp.flip(s1, axis=t + 1)
    out = jnp.concatenate([s0, s1], axis=c + 1)
    return out.reshape(B, -1)


def _circuit_matrix(coeffs, qmats):
    st = jnp.eye(DIMQ, dtype=jnp.complex64)
    for j in range(QL):
        st = st @ qmats[j].T
        cx = coeffs[j, 0]
        cy = coeffs[j, 1]
        rx = jnp.stack([jnp.stack([jnp.cos(cx / 2) + 0j, -1j * jnp.sin(cx / 2)]),
                        jnp.stack([-1j * jnp.sin(cx / 2), jnp.cos(cx / 2) + 0j])])
        ry = jnp.stack([jnp.stack([jnp.cos(cy / 2) + 0j, -jnp.sin(cy / 2) + 0j]),
                        jnp.stack([jnp.sin(cy / 2) + 0j, jnp.cos(cy / 2) + 0j])])
        for i in range(NQ):
            st = _apply_1q(st, rx, i)
        for i in range(NQ):
            st = _apply_1q(st, ry, i)
        for i in range(NQ - 1):
            st = _apply_cnot(st, i, i + 1)
        st = _apply_cnot(st, NQ - 1, 0)
    return st


def _cayley_batch(Ar, Ai):
    A = Ar.astype(jnp.complex64) + 1j * Ai.astype(jnp.complex64)
    A = A + jnp.conj(jnp.swapaxes(A, -1, -2))
    I = jnp.eye(A.shape[-1], dtype=A.dtype)
    return jnp.einsum('...ij,...jk->...ik', A - 1j * I, jnp.linalg.inv(A + 1j * I))


def _edge_mlp_body(u0_ref, u1_ref, rad_ref, att_ref, wr_ref, wa_ref, b1_ref,
                   w2t_ref, b2_ref, m_ref):
    u = u0_ref[...] + u1_ref[...]
    rad = rad_ref[...]
    att = att_ref[...]
    uu = u + rad * wr_ref[...] + att * wa_ref[...] + b1_ref[...]
    z = _silu(uu)
    m_ref[...] = _silu(
        jnp.dot(z, w2t_ref[...], preferred_element_type=jnp.float32)
        + b2_ref[...])


def _edge_mlp(u0, u1, rad, att, wr, wa, b1, w2t, b2):
    grid = (E // BE,)
    return pl.pallas_call(
        _edge_mlp_body,
        grid=grid,
        in_specs=[
            pl.BlockSpec((BE, H), lambda i: (i, 0)),
            pl.BlockSpec((BE, H), lambda i: (i, 0)),
            pl.BlockSpec((BE, 1), lambda i: (i, 0)),
            pl.BlockSpec((BE, 1), lambda i: (i, 0)),
            pl.BlockSpec((1, H), lambda i: (0, 0)),
            pl.BlockSpec((1, H), lambda i: (0, 0)),
            pl.BlockSpec((1, H), lambda i: (0, 0)),
            pl.BlockSpec((H, H), lambda i: (0, 0)),
            pl.BlockSpec((1, H), lambda i: (0, 0)),
        ],
        out_specs=pl.BlockSpec((BE, H), lambda i: (i, 0)),
        out_shape=jax.ShapeDtypeStruct((E, H), jnp.float32),
    )(u0, u1, rad, att, wr, wa, b1, w2t, b2)


def _coord_mlp_body(u0_ref, u1_ref, rad_ref, att_ref, cd_ref, wr_ref, wa_ref,
                    b1_ref, w2t_ref, b2_ref, w3_ref, tr_ref):
    u = u0_ref[...] + u1_ref[...]
    rad = rad_ref[...]
    att = att_ref[...]
    uu = u + rad * wr_ref[...] + att * wa_ref[...] + b1_ref[...]
    z = _silu(uu)
    z2 = _silu(jnp.dot(z, w2t_ref[...], preferred_element_type=jnp.float32)
               + b2_ref[...])
    t = jnp.dot(z2, w3_ref[...], preferred_element_type=jnp.float32)
    tr_ref[...] = cd_ref[...] * t


def _coord_mlp(u0, u1, rad, att, cd, wr, wa, b1, w2t, b2, w3):
    grid = (E // BE,)
    return pl.pallas_call(
        _coord_mlp_body,
        grid=grid,
        in_specs=[
            pl.BlockSpec((BE, H), lambda i: (i, 0)),
            pl.BlockSpec((BE, H), lambda i: (i, 0)),
            pl.BlockSpec((BE, 1), lambda i: (i, 0)),
            pl.BlockSpec((BE, 1), lambda i: (i, 0)),
            pl.BlockSpec((BE, H), lambda i: (i, 0)),
            pl.BlockSpec((1, H), lambda i: (0, 0)),
            pl.BlockSpec((1, H), lambda i: (0, 0)),
            pl.BlockSpec((1, H), lambda i: (0, 0)),
            pl.BlockSpec((H, H), lambda i: (0, 0)),
            pl.BlockSpec((1, H), lambda i: (0, 0)),
            pl.BlockSpec((H, 1), lambda i: (0, 0)),
        ],
        out_specs=pl.BlockSpec((BE, H), lambda i: (i, 0)),
        out_shape=jax.ShapeDtypeStruct((E, H), jnp.float32),
    )(u0, u1, rad, att, cd, wr, wa, b1, w2t, b2, w3)


def _node_update_body(hh_ref, agg0_ref, agg1_ref, a1_ref, a2_ref, encb_ref,
                      d_ref, decb_ref, wrn_ref, wcn_ref, out_ref, pr_ref,
                      pc_ref):
    hh = hh_ref[...]
    agg = (agg0_ref[0] + agg1_ref[0]) * (1.0 / NF)
    q = (jnp.dot(hh, a1_ref[...], preferred_element_type=jnp.float32)
         + jnp.dot(agg, a2_ref[...], preferred_element_type=jnp.float32)
         + encb_ref[...])
    s = jnp.sum(q * q, axis=1, keepdims=True)
    normed = q * lax.rsqrt(s + 1e-12)
    out = hh + jnp.dot(normed, d_ref[...],
                       preferred_element_type=jnp.float32) + decb_ref[...]
    out_ref[...] = out
    pr_ref[...] = jnp.dot(out, wrn_ref[...], preferred_element_type=jnp.float32)
    pc_ref[...] = jnp.dot(out, wcn_ref[...], preferred_element_type=jnp.float32)


def _node_update(hh, parts, a1, a2, encb, d, decb, wrn, wcn):
    grid = (N // BN,)
    return pl.pallas_call(
        _node_update_body,
        grid=grid,
        in_specs=[
            pl.BlockSpec((BN, H), lambda i: (i, 0)),
            pl.BlockSpec((1, BN, H), lambda i: (0, i, 0)),
            pl.BlockSpec((1, BN, H), lambda i: (1, i, 0)),
            pl.BlockSpec((H, H), lambda i: (0, 0)),
            pl.BlockSpec((H, H), lambda i: (0, 0)),
            pl.BlockSpec((1, H), lambda i: (0, 0)),
            pl.BlockSpec((H, H), lambda i: (0, 0)),
            pl.BlockSpec((1, H), lambda i: (0, 0)),
            pl.BlockSpec((H, H), lambda i: (0, 0)),
            pl.BlockSpec((H, H), lambda i: (0, 0)),
        ],
        out_specs=[
            pl.BlockSpec((BN, H), lambda i: (i, 0)),
            pl.BlockSpec((BN, H), lambda i: (i, 0)),
            pl.BlockSpec((BN, H), lambda i: (i, 0)),
        ],
        out_shape=[
            jax.ShapeDtypeStruct((N, H), jnp.float32),
            jax.ShapeDtypeStruct((N, H), jnp.float32),
            jax.ShapeDtypeStruct((N, H), jnp.float32),
        ],
    )(hh, parts, parts, a1, a2, encb, d, decb, wrn, wcn)


def _project_body(hh_ref, wr_ref, wc_ref, pr_ref, pc_ref):
    hh = hh_ref[...]
    pr_ref[...] = jnp.dot(hh, wr_ref[...], preferred_element_type=jnp.float32)
    pc_ref[...] = jnp.dot(hh, wc_ref[...], preferred_element_type=jnp.float32)


def _project(hh, wr, wc):
    grid = (N // BN,)
    return pl.pallas_call(
        _project_body,
        grid=grid,
        in_specs=[
            pl.BlockSpec((BN, H), lambda i: (i, 0)),
            pl.BlockSpec((H, H), lambda i: (0, 0)),
            pl.BlockSpec((H, H), lambda i: (0, 0)),
        ],
        out_specs=[
            pl.BlockSpec((BN, H), lambda i: (i, 0)),
            pl.BlockSpec((BN, H), lambda i: (i, 0)),
        ],
        out_shape=[
            jax.ShapeDtypeStruct((N, H), jnp.float32),
            jax.ShapeDtypeStruct((N, H), jnp.float32),
        ],
    )(hh, wr, wc)


def _wid():
    return lax.axis_index("s") * 2 + lax.axis_index("c")


@functools.partial(
    pl.kernel, mesh=_SC_MESH,
    out_type=jax.ShapeDtypeStruct((2, NW, NCH, CH, H), jnp.float32),
    scratch_types=[
        pltpu.VMEM((NCH, CH), jnp.int32),
        pltpu.VMEM((NCH, CH), jnp.int32),
        pltpu.VMEM((CH, H), jnp.float32),
        pltpu.VMEM((CH, H), jnp.float32),
        pltpu.SemaphoreType.DMA,
        pltpu.SemaphoreType.DMA,
    ])
def _sc_gather2(pr_hbm, pc_hbm, row_hbm, col_hbm, u_hbm, idxr, idxc,
                bufa, bufb, sem, semo):
    w = _wid()
    pltpu.sync_copy(row_hbm.at[w], idxr)
    pltpu.sync_copy(col_hbm.at[w], idxc)

    def step(j, carry):
        da = pltpu.async_copy(pr_hbm.at[idxr.at[j]], bufa, sem)
        db = pltpu.async_copy(pc_hbm.at[idxc.at[j]], bufb, sem)
        da.wait()
        db.wait()
        oa = pltpu.async_copy(bufa, u_hbm.at[0, w, j], semo)
        ob = pltpu.async_copy(bufb, u_hbm.at[1, w, j], semo)
        oa.wait()
        ob.wait()
        return carry

    lax.fori_loop(0, NCH, step, 0)


@functools.partial(
    pl.kernel, mesh=_SC_MESH,
    out_type=jax.ShapeDtypeStruct((2, NSUB, NCH2, CH2, H), jnp.float32),
    scratch_types=[
        pltpu.VMEM((NCH2, CH2), jnp.int32),
        pltpu.VMEM((CH2, H), jnp.float32),
        pltpu.VMEM_SHARED((N, H), jnp.float32),
        pltpu.SemaphoreType.DMA,
    ])
def _sc_gather_split(tbl_hbm, idx_hbm, u_hbm, idx, buf, tblmem, sem):
    cid = lax.axis_index("c")
    sid = lax.axis_index("s")
    pltpu.sync_copy(tbl_hbm.at[cid, pl.ds(sid * NLD, NLD)],
                    tblmem.at[pl.ds(sid * NLD, NLD)])
    pltpu.sync_copy(idx_hbm.at[cid, sid], idx)
    plsc.subcore_barrier()

    def step(j, c):
        pltpu.sync_copy(tblmem.at[idx.at[j]], buf)
        pltpu.sync_copy(buf, u_hbm.at[cid, sid, j])
        return c

    lax.fori_loop(0, NCH2, step, 0)


def _make_sc_scatter(width):
    @functools.partial(
        pl.kernel, mesh=_SC_MESH,
        out_type=jax.ShapeDtypeStruct((2, NP, width), jnp.float32),
        scratch_types=[
            pltpu.VMEM((NCH, CH), jnp.int32),
            pltpu.VMEM((CH, width), jnp.float32),
            pltpu.VMEM((ZR, width), jnp.float32),
            pltpu.VMEM_SHARED((NP, width), jnp.float32),
            pltpu.SemaphoreType.DMA,
        ])
    def _sc_scatter(m_hbm, row_hbm, out_hbm, idx, mbuf, zbuf, acc, sem):
        cid = lax.axis_index("c")
        sid = lax.axis_index("s")
        w = sid * 2 + cid

        def zrow(i, c):
            for k in range(width // 16):
                zbuf[i, pl.ds(k * 16, 16)] = jnp.zeros((16,), jnp.float32)
            return c

        lax.fori_loop(0, ZR, zrow, 0)

        def zcp(t, c):
            pltpu.sync_copy(zbuf, acc.at[pl.ds(sid * NPS + t * ZR, ZR)])
            return c

        lax.fori_loop(0, NPS // ZR, zcp, 0)
        plsc.subcore_barrier()

        pltpu.sync_copy(row_hbm.at[w], idx)

        def step(j, c):
            pltpu.sync_copy(m_hbm.at[w, j], mbuf)
            pltpu.sync_copy(mbuf, acc.at[idx.at[j]], add=True)
            return c

        lax.fori_loop(0, NCH, step, 0)
        plsc.subcore_barrier()
        pltpu.sync_copy(acc.at[pl.ds(sid * NPS, NPS)],
                        out_hbm.at[cid, pl.ds(sid * NPS, NPS)])

    return _sc_scatter


_sc_scatter_h = _make_sc_scatter(H)


def _geom_body(d0_ref, d1_ref, rad_ref, cd_ref):
    d = d0_ref[...] + d1_ref[...]
    radial = jnp.sum(d * d, axis=1, keepdims=True)
    rad_ref[...] = radial
    cd_ref[...] = d / (jnp.sqrt(radial + 1e-8) + 1.0)


def _geometry(d0, d1):
    grid = (E // BE,)
    return pl.pallas_call(
        _geom_body,
        grid=grid,
        in_specs=[
            pl.BlockSpec((BE, H), lambda i: (i, 0)),
            pl.BlockSpec((BE, H), lambda i: (i, 0)),
        ],
        out_specs=[
            pl.BlockSpec((BE, 1), lambda i: (i, 0)),
            pl.BlockSpec((BE, H), lambda i: (i, 0)),
        ],
        out_shape=[
            jax.ShapeDtypeStruct((E, 1), jnp.float32),
            jax.ShapeDtypeStruct((E, H), jnp.float32),
        ],
    )(d0, d1)


def kernel(h, x, edge_index, edge_attr, edge_w1, edge_b1, edge_w2, edge_b2,
           enc_w, enc_b, coeffs, A_real, A_imag, dec_w, dec_b, cm_w1, cm_b1,
           cm_w2, cm_b2, cm_w3):
    row = edge_index[0]
    col = edge_index[1]

    qm = _cayley_batch(A_real.reshape(NLAYERS * QL, DIMQ, DIMQ),
                       A_imag.reshape(NLAYERS * QL, DIMQ, DIMQ))
    qm = qm.reshape(NLAYERS, QL, DIMQ, DIMQ)
    Ds = []
    for i in range(NLAYERS):
        CM = _circuit_matrix(coeffs[i], [qm[i, j] for j in range(QL)])
        Ds.append(jnp.real(CM) @ dec_w[i].T)

    row3 = row.reshape(NW, NCH, CH)
    col3 = col.reshape(NW, NCH, CH)
    xpad = jnp.pad(x, ((0, 0), (0, H - x.shape[1])))
    d2 = _sc_gather2(xpad, -xpad, row3, col3).reshape(2, E, H)
    radial, cd = _geometry(d2[0], d2[1])

    hh = h
    pr, pc = _project(hh, edge_w1[0][:, :H].T, edge_w1[0][:, H:2 * H].T)
    for i in range(NLAYERS):
        ew1 = edge_w1[i]
        u2 = _sc_gather2(pr, pc, row3, col3).reshape(2, E, H)
        m = _edge_mlp(u2[0], u2[1], radial, edge_attr,
                      ew1[:, 2 * H].reshape(1, H), ew1[:, 2 * H + 1].reshape(1, H),
                      edge_b1[i].reshape(1, H), edge_w2[i].T,
                      edge_b2[i].reshape(1, H))
        parts = _sc_scatter_h(m.reshape(NW, NCH, CH, H), row3)
        if i + 1 < NLAYERS:
            nw1 = edge_w1[i + 1]
            wrn, wcn = nw1[:, :H].T, nw1[:, H:2 * H].T
        else:
            wrn, wcn = cm_w1[:, :H].T, cm_w1[:, H:2 * H].T
        hh, pr, pc = _node_update(
            hh, parts, enc_w[i][:, :H].T, enc_w[i][:, H:].T,
            enc_b[i].reshape(1, H), Ds[i], dec_b[i].reshape(1, H), wrn, wcn)

    u2 = _sc_gather2(pr, pc, row3, col3).reshape(2, E, H)
    trans = _coord_mlp(u2[0], u2[1], radial, edge_attr, cd,
                       cm_w1[:, 2 * H].reshape(1, H),
                       cm_w1[:, 2 * H + 1].reshape(1, H),
                       cm_b1.reshape(1, H), cm_w2.T, cm_b2.reshape(1, H),
                       cm_w3.T)
    partsx = _sc_scatter_h(trans.reshape(NW, NCH, CH, H), row3)
    aggc = (partsx[0, :N, :3] + partsx[1, :N, :3]) * (1.0 / NF)
    x_out = x + aggc
    return hh, x_out


# speedup vs baseline: 1.3147x; 1.3147x over previous
import functools

import jax
import jax.numpy as jnp
from jax import lax
from jax.experimental import pallas as pl
from jax.experimental.pallas import tpu as pltpu
from jax.experimental.pallas import tpu_sc as plsc

N = 10000
E = 320000
H = 128
NQ = 7
QL = 2
DIMQ = 2 ** NQ
NF = 100.0
NLAYERS = 2

BE = 2000
BN = 2000

NW = 32
EPW = E // NW
CH = 80
NCH = EPW // CH
NP = 10240
NPS = NP // 16
ZR = 80

NSUB = 16
EPS = E // NSUB
CH2 = 125
NCH2 = EPS // CH2
NLD = N // NSUB

_SC_MESH = plsc.VectorSubcoreMesh(core_axis_name="c", subcore_axis_name="s")


def _silu(v):
    return v * jax.nn.sigmoid(v)


def _apply_1q(st, gate, wire):
    B = st.shape[0]
    s = st.reshape(B, 2 ** wire, 2, -1)
    s = jnp.einsum('ab,slbr->slar', gate, s)
    return s.reshape(B, -1)


def _apply_cnot(st, c, t):
    B = st.shape[0]
    s = st.reshape((B,) + (2,) * NQ)
    s0 = jnp.take(s, jnp.array([0]), axis=c + 1)
    s1 = jnp.take(s, jnp.array([1]), axis=c + 1)
    s1 = jnp.flip(s1, axis=t + 1)
    out = jnp.concatenate([s0, s1], axis=c + 1)
    return out.reshape(B, -1)


def _circuit_matrix(coeffs, qmats):
    st = jnp.eye(DIMQ, dtype=jnp.complex64)
    for j in range(QL):
        st = st @ qmats[j].T
        cx = coeffs[j, 0]
        cy = coeffs[j, 1]
        rx = jnp.stack([jnp.stack([jnp.cos(cx / 2) + 0j, -1j * jnp.sin(cx / 2)]),
                        jnp.stack([-1j * jnp.sin(cx / 2), jnp.cos(cx / 2) + 0j])])
        ry = jnp.stack([jnp.stack([jnp.cos(cy / 2) + 0j, -jnp.sin(cy / 2) + 0j]),
                        jnp.stack([jnp.sin(cy / 2) + 0j, jnp.cos(cy / 2) + 0j])])
        for i in range(NQ):
            st = _apply_1q(st, rx, i)
        for i in range(NQ):
            st = _apply_1q(st, ry, i)
        for i in range(NQ - 1):
            st = _apply_cnot(st, i, i + 1)
        st = _apply_cnot(st, NQ - 1, 0)
    return st


def _cayley_batch(Ar, Ai):
    A = Ar.astype(jnp.complex64) + 1j * Ai.astype(jnp.complex64)
    A = A + jnp.conj(jnp.swapaxes(A, -1, -2))
    I = jnp.eye(A.shape[-1], dtype=A.dtype)
    return jnp.einsum('...ij,...jk->...ik', A - 1j * I, jnp.linalg.inv(A + 1j * I))


def _edge_mlp_body(u_ref, rad_ref, att_ref, wr_ref, wa_ref, b1_ref,
                   w2t_ref, b2_ref, m_ref):
    u = u_ref[...]
    rad = rad_ref[...]
    att = att_ref[...]
    uu = u + rad * wr_ref[...] + att * wa_ref[...] + b1_ref[...]
    z = _silu(uu)
    m_ref[...] = _silu(
        jnp.dot(z, w2t_ref[...], preferred_element_type=jnp.float32)
        + b2_ref[...])


def _edge_mlp(u, rad, att, wr, wa, b1, w2t, b2):
    grid = (E // BE,)
    return pl.pallas_call(
        _edge_mlp_body,
        grid=grid,
        in_specs=[
            pl.BlockSpec((BE, H), lambda i: (i, 0)),
            pl.BlockSpec((BE, 1), lambda i: (i, 0)),
            pl.BlockSpec((BE, 1), lambda i: (i, 0)),
            pl.BlockSpec((1, H), lambda i: (0, 0)),
            pl.BlockSpec((1, H), lambda i: (0, 0)),
            pl.BlockSpec((1, H), lambda i: (0, 0)),
            pl.BlockSpec((H, H), lambda i: (0, 0)),
            pl.BlockSpec((1, H), lambda i: (0, 0)),
        ],
        out_specs=pl.BlockSpec((BE, H), lambda i: (i, 0)),
        out_shape=jax.ShapeDtypeStruct((E, H), jnp.float32),
    )(u, rad, att, wr, wa, b1, w2t, b2)


def _coord_mlp_body(u_ref, rad_ref, att_ref, cd_ref, wr_ref, wa_ref,
                    b1_ref, w2t_ref, b2_ref, w3_ref, tr_ref):
    u = u_ref[...]
    rad = rad_ref[...]
    att = att_ref[...]
    uu = u + rad * wr_ref[...] + att * wa_ref[...] + b1_ref[...]
    z = _silu(uu)
    z2 = _silu(jnp.dot(z, w2t_ref[...], preferred_element_type=jnp.float32)
               + b2_ref[...])
    t = jnp.dot(z2, w3_ref[...], preferred_element_type=jnp.float32)
    tr = cd_ref[...] * t
    tr_ref[...] = jnp.pad(tr, ((0, 0), (0, H - 16)))


def _coord_mlp(u, rad, att, cd, wr, wa, b1, w2t, b2, w3):
    grid = (E // BE,)
    return pl.pallas_call(
        _coord_mlp_body,
        grid=grid,
        in_specs=[
            pl.BlockSpec((BE, H), lambda i: (i, 0)),
            pl.BlockSpec((BE, 1), lambda i: (i, 0)),
            pl.BlockSpec((BE, 1), lambda i: (i, 0)),
            pl.BlockSpec((BE, 16), lambda i: (i, 0)),
            pl.BlockSpec((1, H), lambda i: (0, 0)),
            pl.BlockSpec((1, H), lambda i: (0, 0)),
            pl.BlockSpec((1, H), lambda i: (0, 0)),
            pl.BlockSpec((H, H), lambda i: (0, 0)),
            pl.BlockSpec((1, H), lambda i: (0, 0)),
            pl.BlockSpec((H, 1), lambda i: (0, 0)),
        ],
        out_specs=pl.BlockSpec((BE, H), lambda i: (i, 0)),
        out_shape=jax.ShapeDtypeStruct((E, H), jnp.float32),
    )(u, rad, att, cd, wr, wa, b1, w2t, b2, w3)


def _node_update_body(hh_ref, agg0_ref, agg1_ref, a1_ref, a2_ref, encb_ref,
                      d_ref, decb_ref, wrn_ref, wcn_ref, out_ref, pr_ref,
                      pc_ref):
    hh = hh_ref[...]
    agg = (agg0_ref[0] + agg1_ref[0]) * (1.0 / NF)
    q = (jnp.dot(hh, a1_ref[...], preferred_element_type=jnp.float32)
         + jnp.dot(agg, a2_ref[...], preferred_element_type=jnp.float32)
         + encb_ref[...])
    s = jnp.sum(q * q, axis=1, keepdims=True)
    normed = q * lax.rsqrt(s + 1e-12)
    out = hh + jnp.dot(normed, d_ref[...],
                       preferred_element_type=jnp.float32) + decb_ref[...]
    out_ref[...] = out
    pr_ref[...] = jnp.dot(out, wrn_ref[...], preferred_element_type=jnp.float32)
    pc_ref[...] = jnp.dot(out, wcn_ref[...], preferred_element_type=jnp.float32)


def _node_update(hh, parts, a1, a2, encb, d, decb, wrn, wcn):
    grid = (N // BN,)
    return pl.pallas_call(
        _node_update_body,
        grid=grid,
        in_specs=[
            pl.BlockSpec((BN, H), lambda i: (i, 0)),
            pl.BlockSpec((1, BN, H), lambda i: (0, i, 0)),
            pl.BlockSpec((1, BN, H), lambda i: (1, i, 0)),
            pl.BlockSpec((H, H), lambda i: (0, 0)),
            pl.BlockSpec((H, H), lambda i: (0, 0)),
            pl.BlockSpec((1, H), lambda i: (0, 0)),
            pl.BlockSpec((H, H), lambda i: (0, 0)),
            pl.BlockSpec((1, H), lambda i: (0, 0)),
            pl.BlockSpec((H, H), lambda i: (0, 0)),
            pl.BlockSpec((H, H), lambda i: (0, 0)),
        ],
        out_specs=[
            pl.BlockSpec((BN, H), lambda i: (i, 0)),
            pl.BlockSpec((BN, H), lambda i: (i, 0)),
            pl.BlockSpec((BN, H), lambda i: (i, 0)),
        ],
        out_shape=[
            jax.ShapeDtypeStruct((N, H), jnp.float32),
            jax.ShapeDtypeStruct((N, H), jnp.float32),
            jax.ShapeDtypeStruct((N, H), jnp.float32),
        ],
    )(hh, parts, parts, a1, a2, encb, d, decb, wrn, wcn)


def _project_body(hh_ref, wr_ref, wc_ref, pr_ref, pc_ref):
    hh = hh_ref[...]
    pr_ref[...] = jnp.dot(hh, wr_ref[...], preferred_element_type=jnp.float32)
    pc_ref[...] = jnp.dot(hh, wc_ref[...], preferred_element_type=jnp.float32)


def _project(hh, wr, wc):
    grid = (N // BN,)
    return pl.pallas_call(
        _project_body,
        grid=grid,
        in_specs=[
            pl.BlockSpec((BN, H), lambda i: (i, 0)),
            pl.BlockSpec((H, H), lambda i: (0, 0)),
            pl.BlockSpec((H, H), lambda i: (0, 0)),
        ],
        out_specs=[
            pl.BlockSpec((BN, H), lambda i: (i, 0)),
            pl.BlockSpec((BN, H), lambda i: (i, 0)),
        ],
        out_shape=[
            jax.ShapeDtypeStruct((N, H), jnp.float32),
            jax.ShapeDtypeStruct((N, H), jnp.float32),
        ],
    )(hh, wr, wc)


def _wid():
    return lax.axis_index("s") * 2 + lax.axis_index("c")


def _make_gather2(ow):
    @functools.partial(
        pl.kernel, mesh=_SC_MESH,
        out_type=jax.ShapeDtypeStruct((NW, NCH, CH, ow), jnp.float32),
        scratch_types=[
            pltpu.VMEM((NCH, CH), jnp.int32),
            pltpu.VMEM((NCH, CH), jnp.int32),
            pltpu.VMEM((CH, H), jnp.float32),
            pltpu.VMEM((CH, H), jnp.float32),
            pltpu.VMEM((CH, 16), jnp.float32),
            pltpu.SemaphoreType.DMA,
        ])
    def _gather2(pr_hbm, pc_hbm, row_hbm, col_hbm, u_hbm, idxr, idxc,
                 bufa, bufb, bufc, sem):
        w = _wid()
        pltpu.sync_copy(row_hbm.at[w], idxr)
        pltpu.sync_copy(col_hbm.at[w], idxc)

        def step(j, carry):
            da = pltpu.async_copy(pr_hbm.at[idxr.at[j]], bufa, sem)
            db = pltpu.async_copy(pc_hbm.at[idxc.at[j]], bufb, sem)
            da.wait()
            db.wait()

            if ow == H:
                def arow(i, c):
                    for k in range(H // 16):
                        sl = pl.ds(k * 16, 16)
                        bufa[i, sl] = bufa[i, sl] + bufb[i, sl]
                    return c

                lax.fori_loop(0, CH, arow, 0)
                pltpu.sync_copy(bufa, u_hbm.at[w, j])
            else:
                def arow16(i, c):
                    sl = pl.ds(0, 16)
                    bufc[i, sl] = bufa[i, sl] + bufb[i, sl]
                    return c

                lax.fori_loop(0, CH, arow16, 0)
                pltpu.sync_copy(bufc, u_hbm.at[w, j])
            return carry

        lax.fori_loop(0, NCH, step, 0)

    return _gather2


_sc_gather2 = _make_gather2(H)
_sc_gather2_16 = _make_gather2(16)


@functools.partial(
    pl.kernel, mesh=_SC_MESH,
    out_type=jax.ShapeDtypeStruct((2, NSUB, NCH2, CH2, H), jnp.float32),
    scratch_types=[
        pltpu.VMEM((NCH2, CH2), jnp.int32),
        pltpu.VMEM((CH2, H), jnp.float32),
        pltpu.VMEM_SHARED((N, H), jnp.float32),
        pltpu.SemaphoreType.DMA,
    ])
def _sc_gather_split(tbl_hbm, idx_hbm, u_hbm, idx, buf, tblmem, sem):
    cid = lax.axis_index("c")
    sid = lax.axis_index("s")
    pltpu.sync_copy(tbl_hbm.at[cid, pl.ds(sid * NLD, NLD)],
                    tblmem.at[pl.ds(sid * NLD, NLD)])
    pltpu.sync_copy(idx_hbm.at[cid, sid], idx)
    plsc.subcore_barrier()

    def step(j, c):
        pltpu.sync_copy(tblmem.at[idx.at[j]], buf)
        pltpu.sync_copy(buf, u_hbm.at[cid, sid, j])
        return c

    lax.fori_loop(0, NCH2, step, 0)


def _make_sc_scatter(width):
    @functools.partial(
        pl.kernel, mesh=_SC_MESH,
        out_type=jax.ShapeDtypeStruct((2, NP, width), jnp.float32),
        scratch_types=[
            pltpu.VMEM((NCH, CH), jnp.int32),
            pltpu.VMEM((CH, width), jnp.float32),
            pltpu.VMEM((ZR, width), jnp.float32),
            pltpu.VMEM_SHARED((NP, width), jnp.float32),
            pltpu.SemaphoreType.DMA,
        ])
    def _sc_scatter(m_hbm, row_hbm, out_hbm, idx, mbuf, zbuf, acc, sem):
        cid = lax.axis_index("c")
        sid = lax.axis_index("s")
        w = sid * 2 + cid

        def zrow(i, c):
            for k in range(width // 16):
                zbuf[i, pl.ds(k * 16, 16)] = jnp.zeros((16,), jnp.float32)
            return c

        lax.fori_loop(0, ZR, zrow, 0)

        def zcp(t, c):
            pltpu.sync_copy(zbuf, acc.at[pl.ds(sid * NPS + t * ZR, ZR)])
            return c

        lax.fori_loop(0, NPS // ZR, zcp, 0)
        plsc.subcore_barrier()

        pltpu.sync_copy(row_hbm.at[w], idx)

        def step(j, c):
            pltpu.sync_copy(m_hbm.at[w, j], mbuf)
            pltpu.sync_copy(mbuf, acc.at[idx.at[j]], add=True)
            return c

        lax.fori_loop(0, NCH, step, 0)
        plsc.subcore_barrier()
        pltpu.sync_copy(acc.at[pl.ds(sid * NPS, NPS)],
                        out_hbm.at[cid, pl.ds(sid * NPS, NPS)])

    return _sc_scatter


_sc_scatter_h = _make_sc_scatter(H)
_sc_scatter_16 = _make_sc_scatter(16)


def _geom_body(d_ref, rad_ref, cd_ref):
    d = d_ref[...]
    radial = jnp.sum(d * d, axis=1, keepdims=True)
    rad_ref[...] = radial
    cd_ref[...] = d / (jnp.sqrt(radial + 1e-8) + 1.0)


def _geometry(d):
    grid = (E // BE,)
    return pl.pallas_call(
        _geom_body,
        grid=grid,
        in_specs=[
            pl.BlockSpec((BE, 16), lambda i: (i, 0)),
        ],
        out_specs=[
            pl.BlockSpec((BE, 1), lambda i: (i, 0)),
            pl.BlockSpec((BE, 16), lambda i: (i, 0)),
        ],
        out_shape=[
            jax.ShapeDtypeStruct((E, 1), jnp.float32),
            jax.ShapeDtypeStruct((E, 16), jnp.float32),
        ],
    )(d)


def kernel(h, x, edge_index, edge_attr, edge_w1, edge_b1, edge_w2, edge_b2,
           enc_w, enc_b, coeffs, A_real, A_imag, dec_w, dec_b, cm_w1, cm_b1,
           cm_w2, cm_b2, cm_w3):
    row = edge_index[0]
    col = edge_index[1]

    qm = _cayley_batch(A_real.reshape(NLAYERS * QL, DIMQ, DIMQ),
                       A_imag.reshape(NLAYERS * QL, DIMQ, DIMQ))
    qm = qm.reshape(NLAYERS, QL, DIMQ, DIMQ)
    Ds = []
    for i in range(NLAYERS):
        CM = _circuit_matrix(coeffs[i], [qm[i, j] for j in range(QL)])
        Ds.append(jnp.real(CM) @ dec_w[i].T)

    row3 = row.reshape(NW, NCH, CH)
    col3 = col.reshape(NW, NCH, CH)
    xpad = jnp.pad(x, ((0, 0), (0, H - x.shape[1])))
    d16 = _sc_gather2_16(xpad, -xpad, row3, col3).reshape(E, 16)
    radial, cd = _geometry(d16)

    hh = h
    pr, pc = _project(hh, edge_w1[0][:, :H].T, edge_w1[0][:, H:2 * H].T)
    for i in range(NLAYERS):
        ew1 = edge_w1[i]
        u = _sc_gather2(pr, pc, row3, col3).reshape(E, H)
        m = _edge_mlp(u, radial, edge_attr,
                      ew1[:, 2 * H].reshape(1, H), ew1[:, 2 * H + 1].reshape(1, H),
                      edge_b1[i].reshape(1, H), edge_w2[i].T,
                      edge_b2[i].reshape(1, H))
        parts = _sc_scatter_h(m.reshape(NW, NCH, CH, H), row3)
        if i + 1 < NLAYERS:
            nw1 = edge_w1[i + 1]
            wrn, wcn = nw1[:, :H].T, nw1[:, H:2 * H].T
        else:
            wrn, wcn = cm_w1[:, :H].T, cm_w1[:, H:2 * H].T
        hh, pr, pc = _node_update(
            hh, parts, enc_w[i][:, :H].T, enc_w[i][:, H:].T,
            enc_b[i].reshape(1, H), Ds[i], dec_b[i].reshape(1, H), wrn, wcn)

    u = _sc_gather2(pr, pc, row3, col3).reshape(E, H)
    trans = _coord_mlp(u, radial, edge_attr, cd,
                       cm_w1[:, 2 * H].reshape(1, H),
                       cm_w1[:, 2 * H + 1].reshape(1, H),
                       cm_b1.reshape(1, H), cm_w2.T, cm_b2.reshape(1, H),
                       cm_w3.T)
    partsx = _sc_scatter_h(trans.reshape(NW, NCH, CH, H), row3)
    aggc = (partsx[0, :N, :3] + partsx[1, :N, :3]) * (1.0 / NF)
    x_out = x + aggc
    return hh, x_out
